# baseline (speedup 1.0000x reference)
import numpy as np

import jax
import jax.numpy as jnp
from jax.experimental import pallas as pl
from jax.experimental.pallas import tpu as pltpu

B = 100
N = 101
E = 128
HID = 512
NODES = B * N
GB = 10
STEPS = B // GB
NPHASE = 7
EPS = 1e-5
F32 = jnp.float32
BF = jnp.bfloat16


def _tri_matrix():
    j = np.arange(N, dtype=np.float64)
    dinv = 1.0 / np.sqrt(j + 1.0)
    a = np.tril(np.ones((N, N))) * (dinv[:, None] * dinv[None, :])
    return jnp.asarray(a, dtype=F32)


def _mega_body(u_ref, wcat_ref, a_ref, wg_ref, bg_ref, gam_ref, bet_ref,
               w1_ref, b1_ref, w2_ref, b2_ref, xo_ref, mo_ref,
               xs, ys, sg, qg, sf, qf):
    s = pl.program_id(0)
    phase = s // STEPS
    blk = s % STEPS
    base = blk * GB

    def coefs(s_scr, q_scr, g, b):
        mu = s_scr[...] * (1.0 / NODES)
        var = q_scr[...] * (1.0 / NODES) - mu * mu
        scale = g * jax.lax.rsqrt(var + EPS)
        return scale, b - mu * scale

    def acc(sa, qa, s_scr, q_scr):
        @pl.when(blk == 0)
        def _():
            s_scr[...] = sa
            q_scr[...] = qa

        @pl.when(blk != 0)
        def _():
            s_scr[...] += sa
            q_scr[...] += qa

    def gcn_phase(z_of_j, l, dst):
        wg = wg_ref[l]
        bgv = bg_ref[l]
        amat = a_ref[...]
        sa = jnp.zeros((1, E), F32)
        qa = jnp.zeros((1, E), F32)
        for j in range(GB):
            z = z_of_j(j)
            h = jnp.dot(z, wg, preferred_element_type=F32)
            m = jnp.dot(amat, h, preferred_element_type=F32)
            y = z + m + bgv
            dst[base + j] = y
            sa = sa + jnp.sum(y, axis=0, keepdims=True)
            qa = qa + jnp.sum(y * y, axis=0, keepdims=True)
        acc(sa, qa, sg, qg)

    def ff_phase(l, src, dst):
        scale, shift = coefs(sg, qg, gam_ref[l], bet_ref[l])
        w1 = w1_ref[l]
        b1 = b1_ref[l]
        w2 = w2_ref[l]
        b2 = b2_ref[l]
        sa = jnp.zeros((1, E), F32)
        qa = jnp.zeros((1, E), F32)
        for j in range(GB):
            z = src[base + j] * scale + shift
            h1 = jnp.maximum(
                jnp.dot(z.astype(BF), w1, preferred_element_type=F32) + b1,
                0.0)
            t = z + jnp.dot(h1.astype(BF), w2,
                            preferred_element_type=F32) + b2
            dst[base + j] = t
            sa = sa + jnp.sum(t, axis=0, keepdims=True)
            qa = qa + jnp.sum(t * t, axis=0, keepdims=True)
        acc(sa, qa, sf, qf)

    @pl.when(phase == 0)
    def _():
        wcat = wcat_ref[...]
        gcn_phase(
            lambda j: jnp.dot(u_ref[base + j], wcat,
                              preferred_element_type=F32), 0, ys)

    @pl.when(phase == 1)
    def _():
        ff_phase(0, ys, xs)

    @pl.when(phase == 2)
    def _():
        scale, shift = coefs(sf, qf, gam_ref[0], bet_ref[0])
        gcn_phase(lambda j: xs[base + j] * scale + shift, 1, ys)

    @pl.when(phase == 3)
    def _():
        ff_phase(1, ys, xs)

    @pl.when(phase == 4)
    def _():
        scale, shift = coefs(sf, qf, gam_ref[1], bet_ref[1])
        gcn_phase(lambda j: xs[base + j] * scale + shift, 2, ys)

    @pl.when(phase == 5)
    def _():
        ff_phase(2, ys, xs)

    @pl.when(phase == 6)
    def _():
        scale, shift = coefs(sf, qf, gam_ref[2], bet_ref[2])
        for j in range(GB):
            z = xs[base + j] * scale + shift
            xo_ref[j] = z
            mo_ref[j] = jnp.sum(z, axis=0, keepdims=True) * (1.0 / N)


def _const_spec(shape):
    nd = len(shape)
    return pl.BlockSpec(shape, lambda s: (0,) * nd)


def kernel(depot_xy, customer_xy, demand, params):
    z1 = jnp.zeros((B, 1), F32)
    o1 = jnp.ones((B, 1), F32)
    row0 = jnp.concatenate([depot_xy, jnp.zeros((B, 3), F32), o1, z1],
                           axis=1)[:, None, :]
    zc = jnp.zeros((B, N - 1, 1), F32)
    oc = jnp.ones((B, N - 1, 1), F32)
    custp = jnp.concatenate(
        [jnp.zeros((B, N - 1, 2), F32), customer_xy, demand[..., None],
         zc, oc], axis=2)
    u = jnp.concatenate([row0, custp], axis=1)
    wcat = jnp.concatenate(
        [params["Wd"], params["Wi"],
         params["bd"][None, :], params["bi"][None, :]], axis=0)

    layers = params["layers"]
    wgS = jnp.stack([lp["Wg"] for lp in layers])
    bgS = jnp.stack([lp["bg"].reshape(1, E) for lp in layers])
    gamS = jnp.stack([lp["gamma"].reshape(1, E) for lp in layers])
    betS = jnp.stack([lp["beta"].reshape(1, E) for lp in layers])
    w1S = jnp.stack([lp["W1"] for lp in layers]).astype(BF)
    b1S = jnp.stack([lp["b1"].reshape(1, HID) for lp in layers])
    w2S = jnp.stack([lp["W2"] for lp in layers]).astype(BF)
    b2S = jnp.stack([lp["b2"].reshape(1, E) for lp in layers])

    out_map = lambda s: (jnp.maximum(s - (NPHASE - 1) * STEPS, 0), 0, 0)
    xf, mf = pl.pallas_call(
        _mega_body,
        grid=(NPHASE * STEPS,),
        in_specs=[_const_spec((B, N, 7)), _const_spec((7, E)),
                  _const_spec((N, N)),
                  _const_spec((3, E, E)), _const_spec((3, 1, E)),
                  _const_spec((3, 1, E)), _const_spec((3, 1, E)),
                  _const_spec((3, E, HID)), _const_spec((3, 1, HID)),
                  _const_spec((3, HID, E)), _const_spec((3, 1, E))],
        out_specs=[pl.BlockSpec((GB, N, E), out_map),
                   pl.BlockSpec((GB, 1, E), out_map)],
        out_shape=[jax.ShapeDtypeStruct((B, N, E), F32),
                   jax.ShapeDtypeStruct((B, 1, E), F32)],
        scratch_shapes=[pltpu.VMEM((B, N, E), F32), pltpu.VMEM((B, N, E), F32),
                        pltpu.VMEM((1, E), F32), pltpu.VMEM((1, E), F32),
                        pltpu.VMEM((1, E), F32), pltpu.VMEM((1, E), F32)],
    )(u, wcat, _tri_matrix(), wgS, bgS, gamS, betS, w1S, b1S,
      w2S, b2S)

    return xf, mf.reshape(B, E)


# speedup vs baseline: 1.0465x; 1.0465x over previous
import numpy as np

import jax
import jax.numpy as jnp
from jax.experimental import pallas as pl
from jax.experimental.pallas import tpu as pltpu

B = 100
N = 101
E = 128
HID = 512
NODES = B * N
GB = 10
STEPS = B // GB
EPS = 1e-5
F32 = jnp.float32
BF = jnp.bfloat16


def _tri_matrix():
    j = np.arange(N, dtype=np.float64)
    dinv = 1.0 / np.sqrt(j + 1.0)
    a = np.tril(np.ones((N, N))) * (dinv[:, None] * dinv[None, :])
    return jnp.asarray(a, dtype=F32)


def _body(u_ref, wcat_ref, a_ref, wg_ref, bg_ref, gam_ref, bet_ref,
          w1_ref, b1_ref, w2_ref, b2_ref, xo_ref, mo_ref, xs, ys):
    z2 = (jnp.zeros((1, E), F32), jnp.zeros((1, E), F32))

    def coefs(stats, g, b):
        mu = stats[0] * (1.0 / NODES)
        var = stats[1] * (1.0 / NODES) - mu * mu
        scale = g * jax.lax.rsqrt(var + EPS)
        return scale, b - mu * scale

    def gcn_phase(z_of, l, dst):
        wg = wg_ref[l]
        bgv = bg_ref[l]
        amat = a_ref[...]

        def step(i, carry):
            sa, qa = carry
            base = i * GB
            for j in range(GB):
                z = z_of(base + j)
                h = jnp.dot(z, wg, preferred_element_type=F32)
                m = jnp.dot(amat, h, preferred_element_type=F32)
                y = z + m + bgv
                dst[base + j] = y
                sa = sa + jnp.sum(y, axis=0, keepdims=True)
                qa = qa + jnp.sum(y * y, axis=0, keepdims=True)
            return sa, qa

        return jax.lax.fori_loop(0, STEPS, step, z2)

    def ff_phase(stats, l, src, dst):
        scale, shift = coefs(stats, gam_ref[l], bet_ref[l])
        w1 = w1_ref[l]
        b1 = b1_ref[l]
        w2 = w2_ref[l]
        b2 = b2_ref[l]

        def step(i, carry):
            sa, qa = carry
            base = i * GB
            for j in range(GB):
                z = src[base + j] * scale + shift
                h1 = jnp.maximum(
                    jnp.dot(z.astype(BF), w1,
                            preferred_element_type=F32) + b1, 0.0)
                t = z + jnp.dot(h1.astype(BF), w2,
                                preferred_element_type=F32) + b2
                dst[base + j] = t
                sa = sa + jnp.sum(t, axis=0, keepdims=True)
                qa = qa + jnp.sum(t * t, axis=0, keepdims=True)
            return sa, qa

        return jax.lax.fori_loop(0, STEPS, step, z2)

    wcat = wcat_ref[...]
    sg = gcn_phase(lambda g: jnp.dot(u_ref[g], wcat,
                                     preferred_element_type=F32), 0, ys)
    sf = ff_phase(sg, 0, ys, xs)

    for l in (1, 2):
        scale, shift = coefs(sf, gam_ref[l - 1], bet_ref[l - 1])
        sg = gcn_phase(lambda g: xs[g] * scale + shift, l, ys)
        sf = ff_phase(sg, l, ys, xs)

    scale, shift = coefs(sf, gam_ref[2], bet_ref[2])

    def out_step(i, _):
        base = i * GB
        for j in range(GB):
            z = xs[base + j] * scale + shift
            xo_ref[base + j] = z
            mo_ref[base + j] = jnp.sum(z, axis=0, keepdims=True) * (1.0 / N)
        return 0

    jax.lax.fori_loop(0, STEPS, out_step, 0)


def _const_spec(shape):
    nd = len(shape)
    return pl.BlockSpec(shape, lambda: (0,) * nd)


def kernel(depot_xy, customer_xy, demand, params):
    z1 = jnp.zeros((B, 1), F32)
    o1 = jnp.ones((B, 1), F32)
    row0 = jnp.concatenate([depot_xy, jnp.zeros((B, 3), F32), o1, z1],
                           axis=1)[:, None, :]
    zc = jnp.zeros((B, N - 1, 1), F32)
    oc = jnp.ones((B, N - 1, 1), F32)
    custp = jnp.concatenate(
        [jnp.zeros((B, N - 1, 2), F32), customer_xy, demand[..., None],
         zc, oc], axis=2)
    u = jnp.concatenate([row0, custp], axis=1)
    wcat = jnp.concatenate(
        [params["Wd"], params["Wi"],
         params["bd"][None, :], params["bi"][None, :]], axis=0)

    layers = params["layers"]
    wgS = jnp.stack([lp["Wg"] for lp in layers])
    bgS = jnp.stack([lp["bg"].reshape(1, E) for lp in layers])
    gamS = jnp.stack([lp["gamma"].reshape(1, E) for lp in layers])
    betS = jnp.stack([lp["beta"].reshape(1, E) for lp in layers])
    w1S = jnp.stack([lp["W1"] for lp in layers]).astype(BF)
    b1S = jnp.stack([lp["b1"].reshape(1, HID) for lp in layers])
    w2S = jnp.stack([lp["W2"] for lp in layers]).astype(BF)
    b2S = jnp.stack([lp["b2"].reshape(1, E) for lp in layers])

    xf, mf = pl.pallas_call(
        _body,
        in_specs=[_const_spec((B, N, 7)), _const_spec((7, E)),
                  _const_spec((N, N)),
                  _const_spec((3, E, E)), _const_spec((3, 1, E)),
                  _const_spec((3, 1, E)), _const_spec((3, 1, E)),
                  _const_spec((3, E, HID)), _const_spec((3, 1, HID)),
                  _const_spec((3, HID, E)), _const_spec((3, 1, E))],
        out_specs=[_const_spec((B, N, E)), _const_spec((B, 1, E))],
        out_shape=[jax.ShapeDtypeStruct((B, N, E), F32),
                   jax.ShapeDtypeStruct((B, 1, E), F32)],
        scratch_shapes=[pltpu.VMEM((B, N, E), F32),
                        pltpu.VMEM((B, N, E), F32)],
    )(u, wcat, _tri_matrix(), wgS, bgS, gamS, betS, w1S, b1S, w2S, b2S)

    return xf, mf.reshape(B, E)
